# baseline (speedup 1.0000x reference)
import functools

import jax
import jax.numpy as jnp
from jax import lax
from jax.experimental import pallas as pl
from jax.experimental.pallas import tpu as pltpu
from jax.experimental.pallas import tpu_sc as plsc

VOCAB = 100000
EMB = 64
BATCH = 4096
SEQ = 200

NC = 2
NS = 16
NW = NC * NS
LW = BATCH // NW
NBUF = 2
ET = EMB // 8
TILE = LW * EMB

_mesh = plsc.VectorSubcoreMesh(core_axis_name="c", subcore_axis_name="s")


@functools.partial(
    pl.kernel,
    mesh=_mesh,
    out_type=jax.ShapeDtypeStruct((SEQ, ET, NW, 8 * 128), jnp.float32),
    compiler_params=pltpu.CompilerParams(
        use_tc_tiling_on_sc=False, needs_layout_passes=False
    ),
    scratch_types=[
        pltpu.VMEM((SEQ, LW), jnp.int32),
        pltpu.VMEM((NBUF, LW, EMB), jnp.float32),
        pltpu.VMEM((NBUF, TILE), jnp.float32),
        pltpu.SemaphoreType.DMA,
        pltpu.SemaphoreType.DMA,
        pltpu.SemaphoreType.DMA,
        pltpu.SemaphoreType.DMA,
        pltpu.SemaphoreType.DMA,
    ],
)
def _gather_kernel(idx_hbm, table_hbm, out_hbm, idx_v, rows_v, tiles_v,
                   isem, g0, g1, o0, o1):
    gsem = (g0, g1)
    osem = (o0, o1)
    wid = lax.axis_index("s") * NC + lax.axis_index("c")
    lane0 = wid * LW

    cp = pltpu.make_async_copy(idx_hbm.at[:, pl.ds(lane0, LW)], idx_v, isem)
    cp.start()
    cp.wait()

    estride = lax.iota(jnp.int32, 16) * LW

    def gather_desc(s, b):
        return pltpu.make_async_copy(
            table_hbm.at[idx_v.at[s]], rows_v.at[b], gsem[b]
        )

    def wb_descs(s, b):
        return [
            pltpu.make_async_copy(
                tiles_v.at[b, pl.ds(e * 8 * LW, 8 * LW)],
                out_hbm.at[s, e, wid],
                osem[b],
            )
            for e in range(ET)
        ]

    def transpose(b):
        def tloop(l, carry):
            for k in range(EMB // 16):
                v = rows_v[b, l, pl.ds(k * 16, 16)]
                plsc.store_scatter(tiles_v.at[b], [estride + (k * 16 * LW + l)], v)
            return carry

        lax.fori_loop(0, LW, tloop, 0)

    for b in range(NBUF):
        gather_desc(b, b).start()

    def group(g, carry):
        s0 = g * NBUF
        for b in range(NBUF):
            gather_desc(s0 + b, b).wait()
            transpose(b)
            gather_desc(s0 + NBUF + b, b).start()
            for d in wb_descs(s0 + b, b):
                d.start()
        for b in range(NBUF):
            for d in wb_descs(s0 + b, b):
                d.wait()
        return carry

    lax.fori_loop(0, SEQ // NBUF - 1, group, 0)

    s0 = SEQ - NBUF
    for b in range(NBUF):
        gather_desc(s0 + b, b).wait()
        transpose(b)
        for d in wb_descs(s0 + b, b):
            d.start()
    for b in range(NBUF):
        for d in wb_descs(s0 + b, b):
            d.wait()


def kernel(token_ids, table):
    idx_t = token_ids.astype(jnp.int32).T
    out4 = _gather_kernel(idx_t, table)
    out5 = out4.reshape(SEQ, ET, NW, 8, 128)
    out = jnp.transpose(out5, (2, 4, 0, 1, 3))
    return out.reshape(BATCH, SEQ, EMB)
